# baseline (speedup 1.0000x reference)
import functools

import jax
import jax.numpy as jnp
from jax import lax
from jax.experimental import pallas as pl
from jax.experimental.pallas import tpu as pltpu
from jax.experimental.pallas import tpu_sc as plsc


def _add_block(x_ref, pe_ref, o_ref):
    o_ref[...] = x_ref[...] + pe_ref[...]


_D = 768
_NW = 32
_CH = 32
_CHW = _CH * _D
_NVEC = _CHW // 16


def _sc_rows_add(xf, pef):
    nwords = xf.shape[0]
    rows_w = nwords // (_NW * _D)
    nch = rows_w // _CH
    mesh = plsc.VectorSubcoreMesh(core_axis_name="c", subcore_axis_name="s")

    @functools.partial(
        pl.kernel,
        mesh=mesh,
        out_type=jax.ShapeDtypeStruct(xf.shape, xf.dtype),
        scratch_types=[
            pltpu.VMEM((_CHW,), jnp.float32),
            pltpu.VMEM((_CHW,), jnp.float32),
            pltpu.VMEM((_CHW,), jnp.float32),
            pltpu.VMEM((_CHW,), jnp.float32),
            pltpu.SemaphoreType.DMA,
            pltpu.SemaphoreType.DMA,
            pltpu.SemaphoreType.DMA,
            pltpu.SemaphoreType.DMA,
        ],
    )
    def k(x_hbm, pe_hbm, o_hbm, xb0, xb1, pb0, pb1, g0, g1, s0, s1):
        wid = lax.axis_index("c") * 16 + lax.axis_index("s")
        base = wid * (rows_w * _D)
        xb = (xb0, xb1)
        pb = (pb0, pb1)
        gs = (g0, g1)
        ss = (s0, s1)

        def off(c):
            return base + c * _CHW

        gx, gp, sc = {}, {}, {}
        gx[0] = pltpu.async_copy(x_hbm.at[pl.ds(off(0), _CHW)], xb0, g0)
        gp[0] = pltpu.async_copy(pe_hbm.at[pl.ds(off(0), _CHW)], pb0, g0)
        for c in range(nch):
            b = c & 1
            if c + 1 < nch:
                nb = 1 - b
                if c >= 1:
                    sc[c - 1].wait()
                gx[c + 1] = pltpu.async_copy(
                    x_hbm.at[pl.ds(off(c + 1), _CHW)], xb[nb], gs[nb])
                gp[c + 1] = pltpu.async_copy(
                    pe_hbm.at[pl.ds(off(c + 1), _CHW)], pb[nb], gs[nb])
            gx[c].wait()
            gp[c].wait()
            xref, pref = xb[b], pb[b]

            @plsc.parallel_loop(0, _NVEC, unroll=8)
            def _(i):
                sl = pl.ds(i * 16, 16)
                xref[sl] = xref[sl] + pref[sl]

            sc[c] = pltpu.async_copy(xref, o_hbm.at[pl.ds(off(c), _CHW)], ss[b])
        if nch >= 2:
            sc[nch - 2].wait()
        sc[nch - 1].wait()

    return k(xf, pef)


def kernel(x, pos_emb):
    b, s, d = x.shape
    pe = pos_emb[:s]
    n_tc = b - 1
    s_blk = 2048
    out_tc = pl.pallas_call(
        _add_block,
        grid=(s // s_blk, n_tc),
        in_specs=[
            pl.BlockSpec((1, s_blk, d), lambda i, j: (j, i, 0)),
            pl.BlockSpec((s_blk, d), lambda i, j: (i, 0)),
        ],
        out_specs=pl.BlockSpec((1, s_blk, d), lambda i, j: (j, i, 0)),
        out_shape=jax.ShapeDtypeStruct((b, s, d), x.dtype),
    )(x, pe)
    of = _sc_rows_add(x[b - 1].reshape(-1), pe.reshape(-1))
    return lax.dynamic_update_slice(out_tc, of.reshape(1, s, d), (b - 1, 0, 0))


# speedup vs baseline: 3.3424x; 3.3424x over previous
import jax
import jax.numpy as jnp
from jax.experimental import pallas as pl
from jax.experimental.pallas import tpu as pltpu


def _add_kernel(x_ref, pe_ref, o_ref):
    o_ref[...] = x_ref[...] + pe_ref[...]


def kernel(x, pos_emb):
    b, s, d = x.shape
    pe = pos_emb[:s]
    s_blk = 4096
    grid = (s // s_blk, b)
    return pl.pallas_call(
        _add_kernel,
        grid=grid,
        in_specs=[
            pl.BlockSpec((1, s_blk, d), lambda i, j: (j, i, 0)),
            pl.BlockSpec((s_blk, d), lambda i, j: (i, 0)),
        ],
        out_specs=pl.BlockSpec((1, s_blk, d), lambda i, j: (j, i, 0)),
        out_shape=jax.ShapeDtypeStruct((b, s, d), x.dtype),
        compiler_params=pltpu.CompilerParams(vmem_limit_bytes=100 * 1024 * 1024),
    )(x, pe)


# speedup vs baseline: 3.3481x; 1.0017x over previous
import jax
import jax.numpy as jnp
from jax.experimental import pallas as pl
from jax.experimental.pallas import tpu as pltpu


def _add_kernel(xa_ref, xb_ref, pa_ref, pb_ref, o_ref):
    h = xa_ref.shape[1]
    o_ref[0, :h, :] = xa_ref[0] + pa_ref[...]
    o_ref[0, h:, :] = xb_ref[0] + pb_ref[...]


def kernel(x, pos_emb):
    b, s, d = x.shape
    pe = pos_emb[:s]
    h = s // 2
    return pl.pallas_call(
        _add_kernel,
        grid=(b,),
        in_specs=[
            pl.BlockSpec((1, h, d), lambda j: (j, 0, 0)),
            pl.BlockSpec((1, h, d), lambda j: (j, 1, 0)),
            pl.BlockSpec((h, d), lambda j: (0, 0)),
            pl.BlockSpec((h, d), lambda j: (1, 0)),
        ],
        out_specs=pl.BlockSpec((1, s, d), lambda j: (j, 0, 0)),
        out_shape=jax.ShapeDtypeStruct((b, s, d), x.dtype),
        compiler_params=pltpu.CompilerParams(vmem_limit_bytes=100 * 1024 * 1024),
    )(x, x, pe, pe)
